# baseline (speedup 1.0000x reference)
import functools

import jax
import jax.numpy as jnp
from jax import lax
from jax.experimental import pallas as pl
from jax.experimental.pallas import tpu as pltpu
from jax.experimental.pallas import tpu_sc as plsc

_NC = 2
_NS = 16
_L = 16
_RPW_PAD = 32


def _fold_body(emb_ref, w_ref, b_ref, t_ref):
    t_ref[...] = lax.dot_general(
        emb_ref[...], w_ref[...],
        dimension_numbers=(((1,), (1,)), ((), ())),
        preferred_element_type=jnp.float32,
    ) + b_ref[...]


def _fold_table(emb_table, w, b2d):
    return pl.pallas_call(
        _fold_body,
        out_shape=jax.ShapeDtypeStruct((100, 256), jnp.float32),
    )(emb_table, w, b2d)


def _sc_add(t_tab, idx2d, x3):
    B, P, H = x3.shape
    n_workers = _NC * _NS
    rows_per_w = B // n_workers
    mesh = plsc.VectorSubcoreMesh(core_axis_name="c", subcore_axis_name="s")

    @functools.partial(
        pl.kernel,
        out_type=jax.ShapeDtypeStruct((B, P, H), jnp.float32),
        mesh=mesh,
        scratch_types=[
            pltpu.VMEM((_RPW_PAD,), jnp.int32),
            pltpu.VMEM((_RPW_PAD, H), jnp.float32),
            pltpu.VMEM((P, H), jnp.float32),
            pltpu.SemaphoreType.DMA,
        ],
    )
    def run(t_hbm, idx_hbm, x_hbm, out_hbm, idx_loc, t_rows, xbuf, sem):
        wid = lax.axis_index("s") * _NC + lax.axis_index("c")
        base = wid * rows_per_w
        pltpu.sync_copy(idx_hbm.at[wid], idx_loc)
        pltpu.async_copy(t_hbm.at[idx_loc], t_rows, sem).wait()

        def row_step(r, _):
            row = base + r
            pltpu.sync_copy(x_hbm.at[row], xbuf)
            trow = [t_rows[r, pl.ds(c * _L, _L)] for c in range(H // _L)]

            def pos_step(p, _):
                for c in range(H // _L):
                    sl = (p, pl.ds(c * _L, _L))
                    xbuf[sl] = xbuf[sl] + trow[c]
                return 0

            lax.fori_loop(0, P, pos_step, 0, unroll=2)
            pltpu.sync_copy(xbuf, out_hbm.at[row])
            return 0

        lax.fori_loop(0, rows_per_w, row_step, 0)

    return run(t_tab, idx2d, x3)


@jax.jit
def _run(x3, idx2d, emb_table, w, b2d):
    t_tab = _fold_table(emb_table, w, b2d)
    return _sc_add(t_tab, idx2d, x3)


def kernel(x, dts, emb_table, W, b):
    b0, b1, d2, d3, d4 = x.shape
    B = b0 * b1
    P = d2 * d3
    n_workers = _NC * _NS
    rows_per_w = B // n_workers
    x3 = x.reshape(B, P, d4)
    idx = dts.reshape(n_workers, rows_per_w) * 2
    idx2d = jnp.pad(idx, ((0, 0), (0, _RPW_PAD - rows_per_w)))
    out = _run(x3, idx2d, emb_table, W, b.reshape(1, d4))
    return out.reshape(b0, b1, d2, d3, d4)


# speedup vs baseline: 1.4790x; 1.4790x over previous
import functools

import jax
import jax.numpy as jnp
from jax import lax
from jax.experimental import pallas as pl
from jax.experimental.pallas import tpu as pltpu
from jax.experimental.pallas import tpu_sc as plsc

_NC = 2
_NS = 16
_L = 16
_RPW_PAD = 32
_NBUF = 4
_DIST = 2


def _fold_body(emb_ref, w_ref, b_ref, t_ref):
    t_ref[...] = lax.dot_general(
        emb_ref[...], w_ref[...],
        dimension_numbers=(((1,), (1,)), ((), ())),
        preferred_element_type=jnp.float32,
    ) + b_ref[...]


def _fold_table(emb_table, w, b2d):
    return pl.pallas_call(
        _fold_body,
        out_shape=jax.ShapeDtypeStruct((100, 256), jnp.float32),
    )(emb_table, w, b2d)


def _sc_add(t_tab, idx2d, x3):
    B, P, H = x3.shape
    n_workers = _NC * _NS
    rows_per_w = B // n_workers
    n_waves = (rows_per_w + _NBUF - 1) // _NBUF
    mesh = plsc.VectorSubcoreMesh(core_axis_name="c", subcore_axis_name="s")

    @functools.partial(
        pl.kernel,
        out_type=jax.ShapeDtypeStruct((B, P, H), jnp.float32),
        mesh=mesh,
        scratch_types=[
            pltpu.VMEM((_RPW_PAD,), jnp.int32),
            pltpu.VMEM((_RPW_PAD, H), jnp.float32),
            [pltpu.VMEM((P, H), jnp.float32) for _ in range(_NBUF)],
            pltpu.SemaphoreType.DMA,
            [pltpu.SemaphoreType.DMA for _ in range(_NBUF)],
            [pltpu.SemaphoreType.DMA for _ in range(_NBUF)],
        ],
    )
    def run(t_hbm, idx_hbm, x_hbm, out_hbm, idx_loc, t_rows, xbufs,
            sem_t, sems_in, sems_out):
        wid = lax.axis_index("s") * _NC + lax.axis_index("c")
        base = wid * rows_per_w
        pltpu.sync_copy(idx_hbm.at[wid], idx_loc)
        pltpu.async_copy(t_hbm.at[idx_loc], t_rows, sem_t).wait()

        def issue_in(b, r):
            pltpu.async_copy(x_hbm.at[base + r], xbufs[b], sems_in[b])

        def wait_in(b, r):
            pltpu.make_async_copy(x_hbm.at[base + r], xbufs[b], sems_in[b]).wait()

        def issue_out(b, r):
            pltpu.async_copy(xbufs[b], out_hbm.at[base + r], sems_out[b])

        def wait_out(b, r):
            pltpu.make_async_copy(xbufs[b], out_hbm.at[base + r], sems_out[b]).wait()

        for r0 in range(_DIST):
            issue_in(r0 % _NBUF, r0)

        def compute(b, r):
            xb = xbufs[b]
            trow = [t_rows[r, pl.ds(c * _L, _L)] for c in range(H // _L)]

            def pos_step(p, _):
                for c in range(H // _L):
                    plsc.addupdate(xb.at[p, pl.ds(c * _L, _L)], trow[c])
                return 0

            lax.fori_loop(0, P, pos_step, 0, unroll=2)

        def wave(g, _):
            for b in range(_NBUF):
                r = g * _NBUF + b

                @pl.when(r < rows_per_w)
                def _():
                    wait_in(b, r)
                    compute(b, r)
                    issue_out(b, r)
                    nxt = r + _DIST
                    b2 = (b + _DIST) % _NBUF

                    @pl.when(nxt < rows_per_w)
                    def _():
                        @pl.when(r >= _NBUF - _DIST)
                        def _():
                            wait_out(b2, r)
                        issue_in(b2, nxt)
            return 0

        lax.fori_loop(0, n_waves, wave, 0)

        for b in range(_NBUF):
            wait_out(b, 0)

    return run(t_tab, idx2d, x3)


@jax.jit
def _run(x3, idx2d, emb_table, w, b2d):
    t_tab = _fold_table(emb_table, w, b2d)
    return _sc_add(t_tab, idx2d, x3)


def kernel(x, dts, emb_table, W, b):
    b0, b1, d2, d3, d4 = x.shape
    B = b0 * b1
    P = d2 * d3
    n_workers = _NC * _NS
    rows_per_w = B // n_workers
    x3 = x.reshape(B, P, d4)
    idx = dts.reshape(n_workers, rows_per_w) * 2
    idx2d = jnp.pad(idx, ((0, 0), (0, _RPW_PAD - rows_per_w)))
    out = _run(x3, idx2d, emb_table, W, b.reshape(1, d4))
    return out.reshape(b0, b1, d2, d3, d4)


# speedup vs baseline: 1.5053x; 1.0178x over previous
import functools

import jax
import jax.numpy as jnp
from jax import lax
from jax.experimental import pallas as pl
from jax.experimental.pallas import tpu as pltpu
from jax.experimental.pallas import tpu_sc as plsc

_NC = 2
_NS = 16
_RPW_PAD = 32


def _fold_body(emb_ref, w_ref, b_ref, t_ref):
    t_ref[...] = lax.dot_general(
        emb_ref[...], w_ref[...],
        dimension_numbers=(((1,), (1,)), ((), ())),
        preferred_element_type=jnp.float32,
    ) + b_ref[...]


def _fold_table(emb_table, w, b2d):
    return pl.pallas_call(
        _fold_body,
        out_shape=jax.ShapeDtypeStruct((100, 256), jnp.float32),
    )(emb_table, w, b2d)


def _sc_gather(t_tab, idx2d, B, H):
    n_workers = _NC * _NS
    mesh = plsc.VectorSubcoreMesh(core_axis_name="c", subcore_axis_name="s")

    @functools.partial(
        pl.kernel,
        out_type=jax.ShapeDtypeStruct((B, H), jnp.float32),
        mesh=mesh,
        scratch_types=[
            pltpu.VMEM((_RPW_PAD,), jnp.int32),
            pltpu.VMEM((_RPW_PAD, H), jnp.float32),
            pltpu.SemaphoreType.DMA,
        ],
    )
    def run(t_hbm, idx_hbm, out_hbm, idx_loc, t_rows, sem):
        wid = lax.axis_index("s") * _NC + lax.axis_index("c")
        ngroups = 3 + (wid < 4).astype(jnp.int32)
        base = (wid * 3 + jnp.minimum(wid, 4)) * 8
        pltpu.sync_copy(idx_hbm.at[wid], idx_loc)
        pltpu.async_copy(t_hbm.at[idx_loc], t_rows, sem).wait()
        for g in range(4):
            @pl.when(g < ngroups)
            def _():
                off = pl.multiple_of(base + g * 8, 8)
                pltpu.sync_copy(t_rows.at[pl.ds(g * 8, 8)],
                                out_hbm.at[pl.ds(off, 8)])

    return run(t_tab, idx2d)


def _add_body(t_ref, x_ref, o_ref):
    o_ref[...] = x_ref[...] + t_ref[...][:, None, :]


def _tc_add(t_all, x3, rows_per_block):
    B, P, H = x3.shape
    n_blocks = B // rows_per_block
    return pl.pallas_call(
        _add_body,
        grid=(n_blocks,),
        in_specs=[
            pl.BlockSpec((rows_per_block, H), lambda i: (i, 0)),
            pl.BlockSpec((rows_per_block, P, H), lambda i: (i, 0, 0)),
        ],
        out_specs=pl.BlockSpec((rows_per_block, P, H), lambda i: (i, 0, 0)),
        out_shape=jax.ShapeDtypeStruct((B, P, H), jnp.float32),
        compiler_params=pltpu.CompilerParams(
            dimension_semantics=("arbitrary",),
        ),
    )(t_all, x3)


@jax.jit
def _run(x3, idx2d, emb_table, w, b2d):
    B, P, H = x3.shape
    t_tab = _fold_table(emb_table, w, b2d)
    t_all = _sc_gather(t_tab, idx2d, B, H)
    return _tc_add(t_all, x3, rows_per_block=200)


def kernel(x, dts, emb_table, W, b):
    b0, b1, d2, d3, d4 = x.shape
    B = b0 * b1
    P = d2 * d3
    n_workers = _NC * _NS
    x3 = x.reshape(B, P, d4)
    idx_flat = dts.reshape(B) * 2
    rows = []
    for w in range(n_workers):
        nrows = 8 * (3 + (1 if w < 4 else 0))
        base = (w * 3 + min(w, 4)) * 8
        rows.append(jnp.pad(idx_flat[base:base + nrows],
                            (0, _RPW_PAD - nrows)))
    idx2d = jnp.stack(rows)
    out = _run(x3, idx2d, emb_table, W, b.reshape(1, d4))
    return out.reshape(b0, b1, d2, d3, d4)


# speedup vs baseline: 2.9172x; 1.9380x over previous
import functools

import jax
import jax.numpy as jnp
from jax import lax
from jax.experimental import pallas as pl
from jax.experimental.pallas import tpu as pltpu


def _make_body(rows_per_block):
    def _body(idx_ref, emb_ref, w_ref, b_ref, x_ref, o_ref, t_ref):
        i = pl.program_id(0)

        @pl.when(i == 0)
        def _():
            table = lax.dot_general(
                emb_ref[...], w_ref[...],
                dimension_numbers=(((1,), (1,)), ((), ())),
                preferred_element_type=jnp.float32,
            ) + b_ref[...]
            ids = idx_ref[...] * 2
            oh = (ids == lax.broadcasted_iota(jnp.int32, (1, 100), 1))
            t_ref[...] = lax.dot_general(
                oh.astype(jnp.float32), table,
                dimension_numbers=(((1,), (0,)), ((), ())),
                preferred_element_type=jnp.float32,
            )

        t_rows = t_ref[pl.ds(i * rows_per_block, rows_per_block), :]
        o_ref[...] = x_ref[...] + t_rows[:, None, :]

    return _body


@functools.partial(jax.jit, static_argnames=("rows_per_block",))
def _run(x3, dts2d, emb_table, w, b2d, rows_per_block):
    B, P, H = x3.shape
    n_blocks = B // rows_per_block
    return pl.pallas_call(
        _make_body(rows_per_block),
        grid=(n_blocks,),
        in_specs=[
            pl.BlockSpec((B, 1), lambda i: (0, 0)),
            pl.BlockSpec((100, H), lambda i: (0, 0)),
            pl.BlockSpec((H, H), lambda i: (0, 0)),
            pl.BlockSpec((1, H), lambda i: (0, 0)),
            pl.BlockSpec((rows_per_block, P, H), lambda i: (i, 0, 0)),
        ],
        out_specs=pl.BlockSpec((rows_per_block, P, H), lambda i: (i, 0, 0)),
        out_shape=jax.ShapeDtypeStruct((B, P, H), jnp.float32),
        scratch_shapes=[pltpu.VMEM((B, H), jnp.float32)],
        compiler_params=pltpu.CompilerParams(
            dimension_semantics=("arbitrary",),
        ),
    )(dts2d, emb_table, w, b2d, x3)


def kernel(x, dts, emb_table, W, b):
    b0, b1, d2, d3, d4 = x.shape
    B = b0 * b1
    P = d2 * d3
    x3 = x.reshape(B, P, d4)
    dts2d = dts.reshape(B, 1)
    out = _run(x3, dts2d, emb_table, W, b.reshape(1, d4), rows_per_block=160)
    return out.reshape(b0, b1, d2, d3, d4)


# speedup vs baseline: 2.9508x; 1.0115x over previous
import functools

import jax
import jax.numpy as jnp
from jax import lax
from jax.experimental import pallas as pl
from jax.experimental.pallas import tpu as pltpu


def _make_body(rows_per_block):
    def _body(idx_ref, emb_ref, w_ref, b_ref, x_ref, o_ref, t_ref):
        i = pl.program_id(0)

        @pl.when(i == 0)
        def _():
            table = lax.dot_general(
                emb_ref[...], w_ref[...],
                dimension_numbers=(((1,), (1,)), ((), ())),
                preferred_element_type=jnp.float32,
            ) + b_ref[...]
            ids = idx_ref[...] * 2
            oh = (ids == lax.broadcasted_iota(jnp.int32, (1, 100), 1))
            t_ref[...] = lax.dot_general(
                oh.astype(jnp.float32), table,
                dimension_numbers=(((1,), (0,)), ((), ())),
                preferred_element_type=jnp.float32,
            )

        t_rows = t_ref[pl.ds(i * rows_per_block, rows_per_block), :]
        o_ref[...] = x_ref[...] + t_rows[:, None, :]

    return _body


@functools.partial(jax.jit, static_argnames=("rows_per_block",))
def _run(x3, dts2d, emb_table, w, b2d, rows_per_block):
    B, P, H = x3.shape
    n_blocks = B // rows_per_block
    return pl.pallas_call(
        _make_body(rows_per_block),
        grid=(n_blocks,),
        in_specs=[
            pl.BlockSpec((B, 1), lambda i: (0, 0)),
            pl.BlockSpec((100, H), lambda i: (0, 0)),
            pl.BlockSpec((H, H), lambda i: (0, 0)),
            pl.BlockSpec((1, H), lambda i: (0, 0)),
            pl.BlockSpec((rows_per_block, P, H), lambda i: (i, 0, 0)),
        ],
        out_specs=pl.BlockSpec((rows_per_block, P, H), lambda i: (i, 0, 0)),
        out_shape=jax.ShapeDtypeStruct((B, P, H), jnp.float32),
        scratch_shapes=[pltpu.VMEM((B, H), jnp.float32)],
        compiler_params=pltpu.CompilerParams(
            dimension_semantics=("arbitrary",),
        ),
    )(dts2d, emb_table, w, b2d, x3)


def kernel(x, dts, emb_table, W, b):
    b0, b1, d2, d3, d4 = x.shape
    B = b0 * b1
    P = d2 * d3
    x3 = x.reshape(B, P, d4)
    dts2d = dts.reshape(B, 1)
    out = _run(x3, dts2d, emb_table, W, b.reshape(1, d4), rows_per_block=200)
    return out.reshape(b0, b1, d2, d3, d4)
